# baseline (speedup 1.0000x reference)
import jax
import jax.numpy as jnp
from jax import lax
from jax.experimental import pallas as pl
from jax.experimental.pallas import tpu as pltpu
from jax.experimental.pallas import tpu_sc as plsc

BATCH = 16384
HIST = 50
D = 32
NC = 2
NS = 16
NW = NC * NS
ROWS_PW = BATCH // NW
JB = ROWS_PW // 128
NPAIR = HIST // 2


def _transpose_block(g_ref, t_ref, rb):
    def dloop(d, carry):
        dcol = jnp.full((16,), d, jnp.int32)
        for k in range(8):
            v = plsc.load_gather(g_ref, [rb[k], dcol])
            t_ref[d >> 3, d & 7, pl.ds(k * 16, 16)] = v
        return carry

    lax.fori_loop(0, D, dloop, 0)


def _gather_body(table_hbm, idx_hbm, out_hbm, idx_v, idxt_v, g0, g1, t_v,
                 s0, s1):
    wid = lax.axis_index("s") * NC + lax.axis_index("c")
    base = wid * ROWS_PW
    pltpu.sync_copy(idx_hbm.at[pl.ds(base, ROWS_PW)], idx_v)

    iota = lax.iota(jnp.int32, 16)
    rb = [iota + (16 * k) for k in range(32)]

    def hloop(h, carry):
        hcol = jnp.full((16,), h, jnp.int32)
        for k in range(32):
            v = plsc.load_gather(idx_v, [rb[k], hcol])
            idxt_v[h, pl.ds(k * 16, 16)] = v
        return carry

    lax.fori_loop(0, HIST, hloop, 0)

    def fire(h, jj, gbuf, sem):
        pltpu.async_copy(
            table_hbm.at[idxt_v.at[h, pl.ds(128 * jj, 128)]], gbuf, sem
        )

    def drain(gbuf, sem):
        pltpu.make_async_copy(
            table_hbm.at[idxt_v.at[0, pl.ds(0, 128)]], gbuf, sem
        ).wait()

    for jj in range(JB):
        j = wid * JB + jj
        fire(0, jj, g0, s0)
        fire(1, jj, g1, s1)

        def pair(t, carry):
            h0 = 2 * t
            drain(g0, s0)
            _transpose_block(g0, t_v, rb)
            pltpu.sync_copy(t_v, out_hbm.at[h0, :, j])

            @pl.when(h0 + 2 < HIST)
            def _():
                fire(h0 + 2, jj, g0, s0)

            drain(g1, s1)
            _transpose_block(g1, t_v, rb)
            pltpu.sync_copy(t_v, out_hbm.at[h0 + 1, :, j])

            @pl.when(h0 + 3 < HIST)
            def _():
                fire(h0 + 3, jj, g1, s1)

            return carry

        lax.fori_loop(0, NPAIR, pair, 0)


def kernel(input_ids, weight_shard):
    idx = input_ids.astype(jnp.int32)
    mesh = plsc.VectorSubcoreMesh(core_axis_name="c", subcore_axis_name="s")
    out5 = pl.kernel(
        _gather_body,
        out_type=jax.ShapeDtypeStruct((HIST, D // 8, BATCH // 128, 8, 128),
                                      jnp.float32),
        mesh=mesh,
        scratch_types=[
            pltpu.VMEM((ROWS_PW, HIST), jnp.int32),
            pltpu.VMEM((HIST, ROWS_PW), jnp.int32),
            pltpu.VMEM((128, D), jnp.float32),
            pltpu.VMEM((128, D), jnp.float32),
            pltpu.VMEM((D // 8, 8, 128), jnp.float32),
            pltpu.SemaphoreType.DMA,
            pltpu.SemaphoreType.DMA,
        ],
        compiler_params=pltpu.CompilerParams(
            use_tc_tiling_on_sc=False, needs_layout_passes=False
        ),
    )(weight_shard, idx)
    return jnp.transpose(out5, (2, 4, 0, 1, 3)).reshape(BATCH, HIST, D)
